# Initial kernel scaffold; baseline (speedup 1.0000x reference)
import jax
import jax.numpy as jnp
from jax.experimental import pallas as pl


def kernel(token_ids, weight):
    raise NotImplementedError("write your pallas kernel here")


# speedup vs baseline: 1.6839x; 1.6839x over previous
import functools

import jax
import jax.numpy as jnp
from jax import lax
from jax.experimental import pallas as pl
from jax.experimental.pallas import tpu as pltpu
from jax.experimental.pallas import tpu_sc as plsc

NC = 2
NS = 16
NW = NC * NS
CHUNK = 128
EMB = 64


@functools.cache
def _build(n_tokens: int):
    per_w = n_tokens // NW
    nchunk = per_w // CHUNK
    assert per_w * NW == n_tokens and nchunk * CHUNK == per_w

    mesh = plsc.VectorSubcoreMesh(core_axis_name="c", subcore_axis_name="s")

    @functools.partial(
        pl.kernel,
        mesh=mesh,
        out_type=jax.ShapeDtypeStruct((NW, nchunk, CHUNK, EMB), jnp.float32),
        scratch_types=[
            pltpu.VMEM((nchunk, CHUNK), jnp.int32),
            pltpu.VMEM((CHUNK, EMB), jnp.float32),
            pltpu.SemaphoreType.DMA,
        ],
        compiler_params=pltpu.CompilerParams(use_tc_tiling_on_sc=False),
    )
    def emb(tok_hbm, w_hbm, out_hbm, idx_v, rows_v, sem):
        wid = lax.axis_index("s") * NC + lax.axis_index("c")
        pltpu.sync_copy(tok_hbm.at[wid], idx_v)

        def body(c, carry):
            pltpu.async_copy(w_hbm.at[idx_v.at[c]], rows_v, sem).wait()
            pltpu.sync_copy(rows_v, out_hbm.at[wid, c])
            return carry

        lax.fori_loop(0, nchunk, body, 0)

    return emb


def kernel(token_ids, weight):
    s, t = token_ids.shape
    n = s * t
    tok = token_ids.astype(jnp.int32).reshape(NW, n // NW // CHUNK, CHUNK)
    out = _build(n)(tok, weight)
    return out.reshape(s, t, EMB)


# speedup vs baseline: 1.8725x; 1.1120x over previous
import functools

import jax
import jax.numpy as jnp
from jax import lax
from jax.experimental import pallas as pl
from jax.experimental.pallas import tpu as pltpu
from jax.experimental.pallas import tpu_sc as plsc

NC = 2
NS = 16
NW = NC * NS
CHUNK = 128
NBUF = 8
EMB = 64


@functools.cache
def _build(n_tokens: int):
    per_w = n_tokens // NW
    nchunk = per_w // CHUNK
    nouter = nchunk // NBUF
    assert per_w * NW == n_tokens and nchunk * CHUNK == per_w
    assert nouter * NBUF == nchunk and nouter >= 2

    mesh = plsc.VectorSubcoreMesh(core_axis_name="c", subcore_axis_name="s")

    @functools.partial(
        pl.kernel,
        mesh=mesh,
        out_type=jax.ShapeDtypeStruct((NW, nchunk, CHUNK, EMB), jnp.float32),
        scratch_types=(
            [pltpu.VMEM((nchunk, CHUNK), jnp.int32)]
            + [pltpu.VMEM((CHUNK, EMB), jnp.float32) for _ in range(NBUF)]
            + [pltpu.SemaphoreType.DMA for _ in range(2 * NBUF)]
        ),
        compiler_params=pltpu.CompilerParams(use_tc_tiling_on_sc=False),
    )
    def emb(tok_hbm, w_hbm, out_hbm, idx_v, *rest):
        bufs = rest[:NBUF]
        gsem = rest[NBUF:2 * NBUF]
        wsem = rest[2 * NBUF:]
        wid = lax.axis_index("s") * NC + lax.axis_index("c")
        pltpu.sync_copy(tok_hbm.at[wid], idx_v)

        def fire(c, m):
            pltpu.async_copy(w_hbm.at[idx_v.at[c]], bufs[m], gsem[m])

        def drain_gather(c, m):
            pltpu.make_async_copy(out_hbm.at[wid, c], bufs[m], gsem[m]).wait()

        def start_write(c, m):
            pltpu.async_copy(bufs[m], out_hbm.at[wid, c], wsem[m])

        def drain_write(c, m):
            pltpu.make_async_copy(bufs[m], out_hbm.at[wid, c], wsem[m]).wait()

        for m in range(NBUF):
            fire(m, m)

        def body(j, carry):
            c0 = j * NBUF
            for m in range(NBUF):
                drain_gather(c0 + m, m)
                start_write(c0 + m, m)
            for m in range(NBUF):
                drain_write(c0 + m, m)
                fire(c0 + NBUF + m, m)
            return carry

        lax.fori_loop(0, nouter - 1, body, 0)

        c0 = (nouter - 1) * NBUF
        for m in range(NBUF):
            drain_gather(c0 + m, m)
            start_write(c0 + m, m)
        for m in range(NBUF):
            drain_write(c0 + m, m)

    return emb


def kernel(token_ids, weight):
    s, t = token_ids.shape
    n = s * t
    tok = token_ids.astype(jnp.int32).reshape(NW, n // NW // CHUNK, CHUNK)
    out = _build(n)(tok, weight)
    return out.reshape(s, t, EMB)
